# Initial kernel scaffold; baseline (speedup 1.0000x reference)
import jax
import jax.numpy as jnp
from jax.experimental import pallas as pl


def kernel(x, edge_index, W1, b1, W2, b2, fc_W, fc_b):
    raise NotImplementedError("write your pallas kernel here")


# speedup vs baseline: 19.1583x; 19.1583x over previous
import functools

import jax
import jax.numpy as jnp
from jax import lax
from jax.experimental import pallas as pl
from jax.experimental.pallas import tpu as pltpu
from jax.experimental.pallas import tpu_sc as plsc

N = 10000
E = 320000
D_IN = 128
D_HID = 128
D_OUT = 64

NC = 2
NS = 16
B = 128
EROWS = E // B
ROWS_PER_TILE = 79
LAST_ROWS = EROWS - (NC * NS - 1) * ROWS_PER_TILE

NPAD = 10240
NPT = NPAD // NS

BN = 400
GRID = N // BN

def _mesh():
  return plsc.VectorSubcoreMesh(core_axis_name="c", subcore_axis_name="s")


def _deg_body(dst_hbm, out_hbm, ones_v, zb, idx_v, deg_s):
  cid = lax.axis_index("c")
  sid = lax.axis_index("s")
  wid = cid * NS + sid

  def fill_z(i, _):
    zb[pl.ds(i * 16, 16)] = jnp.zeros((16,), jnp.float32)
    return 0

  lax.fori_loop(0, NPT // 16, fill_z, 0)

  def fill_o(i, _):
    ones_v[pl.ds(i * 16, 16)] = jnp.ones((16,), jnp.float32)
    return 0

  lax.fori_loop(0, B // 16, fill_o, 0)

  pltpu.sync_copy(zb.at[pl.ds(0, NPT)], deg_s.at[pl.ds(sid * NPT, NPT)])
  plsc.subcore_barrier()

  row0 = wid * ROWS_PER_TILE
  nb = jnp.where(wid == NC * NS - 1, LAST_ROWS, ROWS_PER_TILE)

  def body(j, _):
    pltpu.sync_copy(dst_hbm.at[row0 + j], idx_v)
    pltpu.sync_copy(ones_v, deg_s.at[idx_v], add=True)
    return 0

  lax.fori_loop(0, nb, body, 0)
  plsc.subcore_barrier()
  pltpu.sync_copy(deg_s.at[pl.ds(sid * NPT, NPT)],
                  out_hbm.at[cid, 0, pl.ds(sid * NPT, NPT)])


def _sc_degree(dst2):
  k = pl.kernel(
      _deg_body,
      out_type=jax.ShapeDtypeStruct((NC, 1, NPAD), jnp.float32),
      mesh=_mesh(),
      scratch_types=[
          pltpu.VMEM((B,), jnp.float32),
          pltpu.VMEM((NPT,), jnp.float32),
          pltpu.VMEM((B,), jnp.int32),
          pltpu.VMEM_SHARED((NPAD,), jnp.float32),
      ],
  )
  return k(dst2)


def _agg_body(g_hbm, src_hbm, dst_hbm, out_hbm,
              stg_a, stg_b, sidx_a, sidx_b, didx_a, didx_b, acc_s,
              gsem_a, gsem_b, D):
  cid = lax.axis_index("c")
  sid = lax.axis_index("s")
  wid = cid * NS + sid

  def fill_z(i, _):
    r = i // (D // 16)
    c = i % (D // 16)
    stg_a[r, pl.ds(c * 16, 16)] = jnp.zeros((16,), jnp.float32)
    return 0

  lax.fori_loop(0, B * (D // 16), fill_z, 0)
  base = sid * NPT
  for kk in range(NPT // B):
    pltpu.sync_copy(stg_a, acc_s.at[pl.ds(base + kk * B, B)])
  rem = NPT % B
  if rem:
    pltpu.sync_copy(stg_a.at[pl.ds(0, rem)],
                    acc_s.at[pl.ds(base + (NPT // B) * B, rem)])
  plsc.subcore_barrier()

  row0 = wid * ROWS_PER_TILE
  nb = jnp.where(wid == NC * NS - 1, LAST_ROWS, ROWS_PER_TILE)

  pltpu.sync_copy(src_hbm.at[row0], sidx_a)
  pltpu.sync_copy(dst_hbm.at[row0], didx_a)
  pltpu.async_copy(g_hbm.at[sidx_a], stg_a, gsem_a)

  def body(j, _):
    even = lax.rem(j, 2) == 0
    @pl.when(even)
    def _():
      pltpu.make_async_copy(g_hbm.at[sidx_a], stg_a, gsem_a).wait()

      @pl.when(j + 1 < nb)
      def _():
        pltpu.sync_copy(src_hbm.at[row0 + j + 1], sidx_b)
        pltpu.sync_copy(dst_hbm.at[row0 + j + 1], didx_b)
        pltpu.async_copy(g_hbm.at[sidx_b], stg_b, gsem_b)

      pltpu.sync_copy(stg_a, acc_s.at[didx_a], add=True)

    @pl.when(jnp.logical_not(even))
    def _():
      pltpu.make_async_copy(g_hbm.at[sidx_b], stg_b, gsem_b).wait()

      @pl.when(j + 1 < nb)
      def _():
        pltpu.sync_copy(src_hbm.at[row0 + j + 1], sidx_a)
        pltpu.sync_copy(dst_hbm.at[row0 + j + 1], didx_a)
        pltpu.async_copy(g_hbm.at[sidx_a], stg_a, gsem_a)

      pltpu.sync_copy(stg_b, acc_s.at[didx_b], add=True)

    return 0

  lax.fori_loop(0, nb, body, 0)
  plsc.subcore_barrier()
  pltpu.sync_copy(acc_s.at[pl.ds(base, NPT)],
                  out_hbm.at[cid, pl.ds(base, NPT)])


def _sc_aggregate(g, src2, dst2, D):
  k = pl.kernel(
      functools.partial(_agg_body, D=D),
      out_type=jax.ShapeDtypeStruct((NC, NPAD, D), jnp.float32),
      mesh=_mesh(),
      compiler_params=pltpu.CompilerParams(use_tc_tiling_on_sc=False),
      scratch_types=[
          pltpu.VMEM((B, D), jnp.float32),
          pltpu.VMEM((B, D), jnp.float32),
          pltpu.VMEM((B,), jnp.int32),
          pltpu.VMEM((B,), jnp.int32),
          pltpu.VMEM((B,), jnp.int32),
          pltpu.VMEM((B,), jnp.int32),
          pltpu.VMEM_SHARED((NPAD, D), jnp.float32),
          pltpu.SemaphoreType.DMA,
          pltpu.SemaphoreType.DMA,
      ],
  )
  return k(g, src2, dst2)


def _tc_dinv_body(degp_ref, dinv_ref):
  deg = degp_ref[0] + degp_ref[1] + 1.0
  dinv_ref[...] = lax.rsqrt(deg)


def _tc_dinv(degp3):
  return pl.pallas_call(
      _tc_dinv_body,
      out_shape=jax.ShapeDtypeStruct((NPAD // 128, 128), jnp.float32),
  )(degp3)


def _tc1_body(dinv_ref, x_ref, w1_ref, g1_ref):
  h = jnp.dot(x_ref[...], w1_ref[...], preferred_element_type=jnp.float32)
  g1_ref[...] = h * dinv_ref[...]


def _tc1(dinv, x, W1):
  return pl.pallas_call(
      _tc1_body,
      grid=(GRID,),
      in_specs=[
          pl.BlockSpec((BN, 1), lambda i: (i, 0)),
          pl.BlockSpec((BN, D_IN), lambda i: (i, 0)),
          pl.BlockSpec((D_IN, D_HID), lambda i: (0, 0)),
      ],
      out_specs=pl.BlockSpec((BN, D_HID), lambda i: (i, 0)),
      out_shape=jax.ShapeDtypeStruct((N, D_HID), jnp.float32),
  )(dinv, x, W1)


def _tc2_body(dinv_ref, accp_ref, g1_ref, w2_ref, b1_ref, g2_ref):
  dinv = dinv_ref[...]
  agg = accp_ref[0] + accp_ref[1] + g1_ref[...]
  t = jnp.maximum(agg * dinv + b1_ref[...], 0.0)
  h2 = jnp.dot(t, w2_ref[...], preferred_element_type=jnp.float32)
  g2_ref[...] = h2 * dinv


def _tc2(dinv, accp, g1, W2, b1):
  return pl.pallas_call(
      _tc2_body,
      grid=(GRID,),
      in_specs=[
          pl.BlockSpec((BN, 1), lambda i: (i, 0)),
          pl.BlockSpec((NC, BN, D_HID), lambda i: (0, i, 0)),
          pl.BlockSpec((BN, D_HID), lambda i: (i, 0)),
          pl.BlockSpec((D_HID, D_OUT), lambda i: (0, 0)),
          pl.BlockSpec((D_HID,), lambda i: (0,)),
      ],
      out_specs=pl.BlockSpec((BN, D_OUT), lambda i: (i, 0)),
      out_shape=jax.ShapeDtypeStruct((N, D_OUT), jnp.float32),
  )(dinv, accp, g1, W2, b1)


def _tc3_body(dinv_ref, accp_ref, g2_ref, b2_ref, fcw_ref, fcb_ref, out_ref):
  dinv = dinv_ref[...]
  agg = accp_ref[0] + accp_ref[1] + g2_ref[...]
  t = agg * dinv + b2_ref[...]
  o = jnp.dot(t, fcw_ref[...], preferred_element_type=jnp.float32)
  out_ref[...] = 1.0 / (1.0 + jnp.exp(-(o + fcb_ref[...])))


def _tc3(dinv, accp, g2, b2, fc_W, fc_b):
  return pl.pallas_call(
      _tc3_body,
      grid=(GRID,),
      in_specs=[
          pl.BlockSpec((BN, 1), lambda i: (i, 0)),
          pl.BlockSpec((NC, BN, D_OUT), lambda i: (0, i, 0)),
          pl.BlockSpec((BN, D_OUT), lambda i: (i, 0)),
          pl.BlockSpec((D_OUT,), lambda i: (0,)),
          pl.BlockSpec((D_OUT, 1), lambda i: (0, 0)),
          pl.BlockSpec((1,), lambda i: (0,)),
      ],
      out_specs=pl.BlockSpec((BN, 1), lambda i: (i, 0)),
      out_shape=jax.ShapeDtypeStruct((N, 1), jnp.float32),
  )(dinv, accp, g2, b2, fc_W, fc_b)


def kernel(x, edge_index, W1, b1, W2, b2, fc_W, fc_b):
  src2 = edge_index[0].astype(jnp.int32).reshape(EROWS, B)
  dst2 = edge_index[1].astype(jnp.int32).reshape(EROWS, B)

  degp = _sc_degree(dst2)
  dinv2 = _tc_dinv(degp.reshape(NC, NPAD // 128, 128))
  dinv = dinv2.reshape(NPAD)[:N].reshape(N, 1)
  g1 = _tc1(dinv, x, W1)
  acc1 = _sc_aggregate(g1, src2, dst2, D_HID)
  g2 = _tc2(dinv, acc1, g1, W2, b1)
  acc2 = _sc_aggregate(g2, src2, dst2, D_OUT)
  return _tc3(dinv, acc2, g2, b2, fc_W, fc_b)


# speedup vs baseline: 35.2644x; 1.8407x over previous
import functools

import jax
import jax.numpy as jnp
from jax import lax
from jax.experimental import pallas as pl
from jax.experimental.pallas import tpu as pltpu
from jax.experimental.pallas import tpu_sc as plsc

N = 10000
E = 320000
D_IN = 128
D_HID = 128
D_OUT = 64

NC = 2
NS = 16
B = 128
EROWS = E // B
ROWS_PER_TILE = 79
LAST_ROWS = EROWS - (NC * NS - 1) * ROWS_PER_TILE

NPAD = 10240
NPT = NPAD // NS

BN = 400
GRID = N // BN

def _mesh():
  return plsc.VectorSubcoreMesh(core_axis_name="c", subcore_axis_name="s")


def _deg_body(ei_hbm, out_hbm, ones_v, zb, didx_all, deg_s, dsem):
  cid = lax.axis_index("c")
  sid = lax.axis_index("s")
  wid = cid * NS + sid

  def fill_z(i, _):
    zb[pl.ds(i * 16, 16)] = jnp.zeros((16,), jnp.float32)
    return 0

  lax.fori_loop(0, NPT // 16, fill_z, 0)

  def fill_o(i, _):
    ones_v[pl.ds(i * 16, 16)] = jnp.ones((16,), jnp.float32)
    return 0

  lax.fori_loop(0, B // 16, fill_o, 0)

  is_last = wid == NC * NS - 1
  nb = jnp.where(is_last, LAST_ROWS, ROWS_PER_TILE)
  off = jnp.where(is_last, ROWS_PER_TILE - LAST_ROWS, 0)
  row0c = wid * ROWS_PER_TILE - off
  pltpu.sync_copy(ei_hbm.at[1, pl.ds(row0c, ROWS_PER_TILE)], didx_all)
  pltpu.sync_copy(zb, deg_s.at[pl.ds(sid * NPT, NPT)])
  plsc.subcore_barrier()

  def body(j, _):
    pltpu.async_copy(ones_v, deg_s.at[didx_all.at[off + j]], dsem, add=True)
    return 0

  lax.fori_loop(0, nb, body, 0)

  def drain(j, _):
    pltpu.make_async_copy(ones_v, deg_s.at[didx_all.at[off]], dsem).wait()
    return 0

  lax.fori_loop(0, nb, drain, 0)
  plsc.subcore_barrier()
  pltpu.sync_copy(deg_s.at[pl.ds(sid * NPT, NPT)],
                  out_hbm.at[cid, 0, pl.ds(sid * NPT, NPT)])


def _sc_degree(ei3):
  k = pl.kernel(
      _deg_body,
      out_type=jax.ShapeDtypeStruct((NC, 1, NPAD), jnp.float32),
      mesh=_mesh(),
      compiler_params=pltpu.CompilerParams(use_tc_tiling_on_sc=False),
      scratch_types=[
          pltpu.VMEM((B,), jnp.float32),
          pltpu.VMEM((NPT,), jnp.float32),
          pltpu.VMEM((ROWS_PER_TILE, B), jnp.int32),
          pltpu.VMEM_SHARED((NPAD,), jnp.float32),
          pltpu.SemaphoreType.DMA,
      ],
  )
  return k(ei3)


def _agg_body(g_hbm, ei_hbm, out_hbm,
              stg0, stg1, isl0, isl1, isl2, acc_s,
              gs0, gs1, ss0, ss1, is0, is1, is2, D):
  stg = (stg0, stg1)
  isl = (isl0, isl1, isl2)
  gsem = (gs0, gs1)
  ssem = (ss0, ss1)
  isem = (is0, is1, is2)
  cid = lax.axis_index("c")
  sid = lax.axis_index("s")
  wid = cid * NS + sid

  def fill_z(i, _):
    r = i // (D // 16)
    c = i % (D // 16)
    stg0[r, pl.ds(c * 16, 16)] = jnp.zeros((16,), jnp.float32)
    return 0

  lax.fori_loop(0, B * (D // 16), fill_z, 0)
  base = sid * NPT
  for kk in range(NPT // B):
    pltpu.sync_copy(stg0, acc_s.at[pl.ds(base + kk * B, B)])

  nb = jnp.where(wid == NC * NS - 1, LAST_ROWS, ROWS_PER_TILE)
  row0 = wid * ROWS_PER_TILE

  pltpu.sync_copy(ei_hbm.at[:, row0], isl0)
  pltpu.async_copy(ei_hbm.at[:, row0 + 1], isl1, is1)
  pltpu.async_copy(g_hbm.at[isl0.at[0]], stg0, gs0)
  plsc.subcore_barrier()

  def group(gidx, _):
    j0 = gidx * 6
    for b in range(6):
      j = j0 + b
      sb = b % 2
      il = b % 3

      @pl.when(j < nb)
      def _():
        @pl.when(j >= 1)
        def _():
          pltpu.make_async_copy(stg[1 - sb], acc_s.at[isl[il].at[1]],
                                ssem[1 - sb]).wait()

        @pl.when(j + 1 < nb)
        def _():
          pltpu.make_async_copy(ei_hbm.at[:, row0], isl[(il + 1) % 3],
                                isem[(il + 1) % 3]).wait()
          pltpu.async_copy(g_hbm.at[isl[(il + 1) % 3].at[0]], stg[1 - sb],
                           gsem[1 - sb])

        @pl.when(j + 2 < nb)
        def _():
          pltpu.async_copy(ei_hbm.at[:, row0 + j + 2], isl[(il + 2) % 3],
                           isem[(il + 2) % 3])

        pltpu.make_async_copy(g_hbm.at[isl[il].at[0]], stg[sb],
                              gsem[sb]).wait()
        pltpu.async_copy(stg[sb], acc_s.at[isl[il].at[1]], ssem[sb],
                         add=True)

    return 0

  lax.fori_loop(0, (ROWS_PER_TILE + 5) // 6, group, 0)
  pltpu.make_async_copy(stg[0], acc_s.at[isl0.at[1]], ssem[0]).wait()

  plsc.subcore_barrier()
  pltpu.sync_copy(acc_s.at[pl.ds(base, NPT)],
                  out_hbm.at[cid, pl.ds(base, NPT)])


def _sc_aggregate(g, ei3, D):
  k = pl.kernel(
      functools.partial(_agg_body, D=D),
      out_type=jax.ShapeDtypeStruct((NC, NPAD, D), jnp.float32),
      mesh=_mesh(),
      compiler_params=pltpu.CompilerParams(use_tc_tiling_on_sc=False),
      scratch_types=(
          [pltpu.VMEM((B, D), jnp.float32) for _ in range(2)]
          + [pltpu.VMEM((2, B), jnp.int32) for _ in range(3)]
          + [pltpu.VMEM_SHARED((NPAD, D), jnp.float32)]
          + [pltpu.SemaphoreType.DMA for _ in range(7)]
      ),
  )
  return k(g, ei3)


def _tc_dinv_body(degp_ref, dinv_ref):
  deg = degp_ref[0] + degp_ref[1] + 1.0
  dinv_ref[...] = lax.rsqrt(deg)


def _tc_dinv(degp3):
  return pl.pallas_call(
      _tc_dinv_body,
      out_shape=jax.ShapeDtypeStruct((NPAD // 128, 128), jnp.float32),
  )(degp3)


def _tc1_body(dinv_ref, x_ref, w1_ref, g1_ref):
  h = jnp.dot(x_ref[...], w1_ref[...], preferred_element_type=jnp.float32)
  g1_ref[...] = h * dinv_ref[...]


def _tc1(dinv, x, W1):
  return pl.pallas_call(
      _tc1_body,
      grid=(GRID,),
      in_specs=[
          pl.BlockSpec((BN, 1), lambda i: (i, 0)),
          pl.BlockSpec((BN, D_IN), lambda i: (i, 0)),
          pl.BlockSpec((D_IN, D_HID), lambda i: (0, 0)),
      ],
      out_specs=pl.BlockSpec((BN, D_HID), lambda i: (i, 0)),
      out_shape=jax.ShapeDtypeStruct((N, D_HID), jnp.float32),
  )(dinv, x, W1)


def _tc2_body(dinv_ref, accp_ref, g1_ref, w2_ref, b1_ref, g2_ref):
  dinv = dinv_ref[...]
  agg = accp_ref[0] + accp_ref[1] + g1_ref[...]
  t = jnp.maximum(agg * dinv + b1_ref[...], 0.0)
  h2 = jnp.dot(t, w2_ref[...], preferred_element_type=jnp.float32)
  g2_ref[...] = h2 * dinv


def _tc2(dinv, accp, g1, W2, b1):
  return pl.pallas_call(
      _tc2_body,
      grid=(GRID,),
      in_specs=[
          pl.BlockSpec((BN, 1), lambda i: (i, 0)),
          pl.BlockSpec((NC, BN, D_HID), lambda i: (0, i, 0)),
          pl.BlockSpec((BN, D_HID), lambda i: (i, 0)),
          pl.BlockSpec((D_HID, D_OUT), lambda i: (0, 0)),
          pl.BlockSpec((D_HID,), lambda i: (0,)),
      ],
      out_specs=pl.BlockSpec((BN, D_OUT), lambda i: (i, 0)),
      out_shape=jax.ShapeDtypeStruct((N, D_OUT), jnp.float32),
  )(dinv, accp, g1, W2, b1)


def _tc3_body(dinv_ref, accp_ref, g2_ref, b2_ref, fcw_ref, fcb_ref, out_ref):
  dinv = dinv_ref[...]
  agg = accp_ref[0] + accp_ref[1] + g2_ref[...]
  t = agg * dinv + b2_ref[...]
  o = jnp.dot(t, fcw_ref[...], preferred_element_type=jnp.float32)
  out_ref[...] = 1.0 / (1.0 + jnp.exp(-(o + fcb_ref[...])))


def _tc3(dinv, accp, g2, b2, fc_W, fc_b):
  return pl.pallas_call(
      _tc3_body,
      grid=(GRID,),
      in_specs=[
          pl.BlockSpec((BN, 1), lambda i: (i, 0)),
          pl.BlockSpec((NC, BN, D_OUT), lambda i: (0, i, 0)),
          pl.BlockSpec((BN, D_OUT), lambda i: (i, 0)),
          pl.BlockSpec((D_OUT,), lambda i: (0,)),
          pl.BlockSpec((D_OUT, 1), lambda i: (0, 0)),
          pl.BlockSpec((1,), lambda i: (0,)),
      ],
      out_specs=pl.BlockSpec((BN, 1), lambda i: (i, 0)),
      out_shape=jax.ShapeDtypeStruct((N, 1), jnp.float32),
  )(dinv, accp, g2, b2, fc_W, fc_b)


def kernel(x, edge_index, W1, b1, W2, b2, fc_W, fc_b):
  ei3 = edge_index.astype(jnp.int32).reshape(2, EROWS, B)

  degp = _sc_degree(ei3)
  dinv2 = _tc_dinv(degp.reshape(NC, NPAD // 128, 128))
  dinv = dinv2.reshape(NPAD)[:N].reshape(N, 1)
  g1 = _tc1(dinv, x, W1)
  acc1 = _sc_aggregate(g1, ei3, D_HID)
  g2 = _tc2(dinv, acc1, g1, W2, b1)
  acc2 = _sc_aggregate(g2, ei3, D_OUT)
  return _tc3(dinv, acc2, g2, b2, fc_W, fc_b)
